# baseline (speedup 1.0000x reference)
import jax
import jax.numpy as jnp
from jax import lax
from jax.experimental import pallas as pl

_N_STEPS = 128
_N1 = 64
_PROP = 0.8
_R = 256


def _body(cam_ref, pts_ref, occ_ref, rd_ref, tmp_ref, out_ref):
    pts = pts_ref[...]
    occ = occ_ref[...]
    tmp = tmp_ref[...]
    R = pts.shape[0]

    lg = jnp.log(1.0 - occ)
    uk = lax.broadcasted_iota(jnp.int32, (_N_STEPS, _N_STEPS), 0)
    um = lax.broadcasted_iota(jnp.int32, (_N_STEPS, _N_STEPS), 1)
    U = (uk < um).astype(jnp.float32)
    s = lax.dot(lg, U, precision=lax.Precision.HIGHEST,
                preferred_element_type=jnp.float32)
    cpr = jnp.exp(s)
    ptsl = pts[:, _N_STEPS - 1 : _N_STEPS]
    o = _PROP * (1.0 - cpr) + (1.0 - _PROP) * (pts / ptsl)
    o = o / o[:, _N_STEPS - 1 : _N_STEPS]

    jq = lax.broadcasted_iota(jnp.int32, (R, _N1), 1).astype(jnp.float32)
    b0 = jq * (1.0 / _N1)
    b1 = (jq + 1.0) * (1.0 / _N1)
    t = tmp * b0 + (1.0 - tmp) * b1

    pos = jnp.zeros((R, _N1), jnp.int32)
    for s in (32, 8, 2):
        f = jnp.zeros((R, _N1), jnp.int32)
        for p in (1, 2, 3):
            v = jnp.take_along_axis(o, pos + (p * s - 1), axis=1)
            f = f + (v <= t).astype(jnp.int32)
        pos = pos + s * f
    v = jnp.take_along_axis(o, pos, axis=1)
    pos = pos + (v <= t).astype(jnp.int32)
    inv = pos + ((pos == _N_STEPS - 1) & (t >= 1.0)).astype(jnp.int32)

    oi_idx = jnp.maximum(inv - 1, 0)
    os_idx = jnp.minimum(inv, _N_STEPS - 1)
    o_inf = jnp.where(inv == 0, -1.0, jnp.take_along_axis(o, oi_idx, axis=1))
    o_sup = jnp.where(inv >= _N_STEPS, 2.0, jnp.take_along_axis(o, os_idx, axis=1))
    d_inf = jnp.take_along_axis(pts, oi_idx, axis=1)
    d_sup = jnp.take_along_axis(pts, os_idx, axis=1)

    denom = o_sup - o_inf
    li = denom > 1e-6
    dist = d_inf + jnp.where(
        li, (t - o_inf) * (d_sup - d_inf) / jnp.where(li, denom, 1.0), 0.0
    )

    m3 = lax.broadcasted_iota(jnp.int32, (R, 3 * _N1), 1)
    dist3 = jnp.take_along_axis(dist, m3 // 3, axis=1)
    c3 = m3 % 3
    rd = rd_ref[...]
    px = cam_ref[0, 0] + dist3 * rd[:, 0:1]
    py = cam_ref[0, 1] + dist3 * rd[:, 1:2]
    pz = cam_ref[0, 2] + dist3 * rd[:, 2:3]
    out_ref[...] = jnp.where(c3 == 0, px, jnp.where(c3 == 1, py, pz))


def kernel(pts_intervals, occ_values, ray_directions, cam_loc, tmp):
    n_rays = pts_intervals.shape[0]
    rd = ray_directions.reshape(n_rays, 3)
    out = pl.pallas_call(
        _body,
        grid=(n_rays // _R,),
        in_specs=[
            pl.BlockSpec((1, 3), lambda i: (0, 0)),
            pl.BlockSpec((_R, _N_STEPS), lambda i: (i, 0)),
            pl.BlockSpec((_R, _N_STEPS), lambda i: (i, 0)),
            pl.BlockSpec((_R, 3), lambda i: (i, 0)),
            pl.BlockSpec((_R, _N1), lambda i: (i, 0)),
        ],
        out_specs=pl.BlockSpec((_R, 3 * _N1), lambda i: (i, 0)),
        out_shape=jax.ShapeDtypeStruct((n_rays, 3 * _N1), jnp.float32),
    )(cam_loc, pts_intervals, occ_values, rd, tmp)
    return out.reshape(n_rays, _N1, 3)


# speedup vs baseline: 1.1662x; 1.1662x over previous
import jax
import jax.numpy as jnp
from jax import lax
from jax.experimental import pallas as pl

_N_STEPS = 128
_N1 = 64
_PROP = 0.8
_R = 512


def _body(cam_ref, pts_ref, occ_ref, rd_ref, tmp_ref, out_ref):
    pts = pts_ref[...]
    occ = occ_ref[...]
    tmp = tmp_ref[...]
    R = pts.shape[0]

    lg = jnp.log(1.0 - occ)
    uk = lax.broadcasted_iota(jnp.int32, (_N_STEPS, _N_STEPS), 0)
    um = lax.broadcasted_iota(jnp.int32, (_N_STEPS, _N_STEPS), 1)
    U = (uk < um).astype(jnp.float32)
    s = lax.dot(lg, U, precision=lax.Precision.HIGHEST,
                preferred_element_type=jnp.float32)
    cpr = jnp.exp(s)
    ptsl = pts[:, _N_STEPS - 1 : _N_STEPS]
    o = _PROP * (1.0 - cpr) + (1.0 - _PROP) * (pts / ptsl)
    o = o / o[:, _N_STEPS - 1 : _N_STEPS]

    jq = lax.broadcasted_iota(jnp.int32, (R, _N1), 1).astype(jnp.float32)
    b0 = jq * (1.0 / _N1)
    b1 = (jq + 1.0) * (1.0 / _N1)
    t = tmp * b0 + (1.0 - tmp) * b1

    pos = jnp.zeros((R, _N1), jnp.int32)
    for step in (64, 32, 16, 8, 4, 2, 1):
        v = jnp.take_along_axis(o, pos + (step - 1), axis=1)
        pos = pos + jnp.where(v <= t, step, 0)
    inv = pos + ((pos == _N_STEPS - 1) & (t >= 1.0)).astype(jnp.int32)

    oi_idx = jnp.maximum(inv - 1, 0)
    os_idx = jnp.minimum(inv, _N_STEPS - 1)
    o_inf = jnp.where(inv == 0, -1.0, jnp.take_along_axis(o, oi_idx, axis=1))
    o_sup = jnp.where(inv >= _N_STEPS, 2.0, jnp.take_along_axis(o, os_idx, axis=1))
    d_inf = jnp.take_along_axis(pts, oi_idx, axis=1)
    d_sup = jnp.take_along_axis(pts, os_idx, axis=1)

    denom = o_sup - o_inf
    li = denom > 1e-6
    dist = d_inf + jnp.where(
        li, (t - o_inf) * (d_sup - d_inf) / jnp.where(li, denom, 1.0), 0.0
    )

    m3 = lax.broadcasted_iota(jnp.int32, (R, 3 * _N1), 1)
    dist3 = jnp.take_along_axis(dist, m3 // 3, axis=1)
    c3 = m3 % 3
    rd = rd_ref[...]
    px = cam_ref[0, 0] + dist3 * rd[:, 0:1]
    py = cam_ref[0, 1] + dist3 * rd[:, 1:2]
    pz = cam_ref[0, 2] + dist3 * rd[:, 2:3]
    out_ref[...] = jnp.where(c3 == 0, px, jnp.where(c3 == 1, py, pz))


def kernel(pts_intervals, occ_values, ray_directions, cam_loc, tmp):
    n_rays = pts_intervals.shape[0]
    rd = ray_directions.reshape(n_rays, 3)
    out = pl.pallas_call(
        _body,
        grid=(n_rays // _R,),
        in_specs=[
            pl.BlockSpec((1, 3), lambda i: (0, 0)),
            pl.BlockSpec((_R, _N_STEPS), lambda i: (i, 0)),
            pl.BlockSpec((_R, _N_STEPS), lambda i: (i, 0)),
            pl.BlockSpec((_R, 3), lambda i: (i, 0)),
            pl.BlockSpec((_R, _N1), lambda i: (i, 0)),
        ],
        out_specs=pl.BlockSpec((_R, 3 * _N1), lambda i: (i, 0)),
        out_shape=jax.ShapeDtypeStruct((n_rays, 3 * _N1), jnp.float32),
    )(cam_loc, pts_intervals, occ_values, rd, tmp)
    return out.reshape(n_rays, _N1, 3)


# speedup vs baseline: 1.2010x; 1.0299x over previous
import jax
import jax.numpy as jnp
from jax import lax
from jax.experimental import pallas as pl

_N_STEPS = 128
_N1 = 64
_PROP = 0.8
_R = 1024


def _body(cam_ref, pts_ref, occ_ref, rd_ref, tmp_ref, out_ref):
    pts = pts_ref[...]
    occ = occ_ref[...]
    tmp = tmp_ref[...]
    R = pts.shape[0]

    lg = jnp.log(1.0 - occ)
    uk = lax.broadcasted_iota(jnp.int32, (_N_STEPS, _N_STEPS), 0)
    um = lax.broadcasted_iota(jnp.int32, (_N_STEPS, _N_STEPS), 1)
    U = (uk < um).astype(jnp.float32)
    s = lax.dot(lg, U, precision=lax.Precision.HIGHEST,
                preferred_element_type=jnp.float32)
    cpr = jnp.exp(s)
    ptsl = pts[:, _N_STEPS - 1 : _N_STEPS]
    o = _PROP * (1.0 - cpr) + (1.0 - _PROP) * (pts / ptsl)
    o = o / o[:, _N_STEPS - 1 : _N_STEPS]

    jq = lax.broadcasted_iota(jnp.int32, (R, _N1), 1).astype(jnp.float32)
    t = (jq + 1.0) * (1.0 / _N1) - tmp * (1.0 / _N1)

    pos = jnp.zeros((R, _N1), jnp.int32)
    for step in (64, 32, 16, 8, 4, 2, 1):
        v = jnp.take_along_axis(o, pos + (step - 1), axis=1)
        pos = pos + jnp.where(v <= t, step, 0)
    inv = pos + ((pos == _N_STEPS - 1) & (t >= 1.0)).astype(jnp.int32)

    oi_idx = jnp.maximum(inv - 1, 0)
    os_idx = jnp.minimum(inv, _N_STEPS - 1)
    o_inf = jnp.where(inv == 0, -1.0, jnp.take_along_axis(o, oi_idx, axis=1))
    o_sup = jnp.where(inv >= _N_STEPS, 2.0, jnp.take_along_axis(o, os_idx, axis=1))
    d_inf = jnp.take_along_axis(pts, oi_idx, axis=1)
    d_sup = jnp.take_along_axis(pts, os_idx, axis=1)

    denom = o_sup - o_inf
    li = denom > 1e-6
    dist = d_inf + jnp.where(
        li, (t - o_inf) * (d_sup - d_inf) / jnp.where(li, denom, 1.0), 0.0
    )

    m3 = lax.broadcasted_iota(jnp.int32, (R, 3 * _N1), 1)
    c3 = m3 - (m3 // 3) * 3
    dist3 = jnp.take_along_axis(dist, m3 // 3, axis=1)
    rd = rd_ref[...]
    px = cam_ref[0, 0] + dist3 * rd[:, 0:1]
    py = cam_ref[0, 1] + dist3 * rd[:, 1:2]
    pz = cam_ref[0, 2] + dist3 * rd[:, 2:3]
    out_ref[...] = jnp.where(c3 == 0, px, jnp.where(c3 == 1, py, pz))


def kernel(pts_intervals, occ_values, ray_directions, cam_loc, tmp):
    n_rays = pts_intervals.shape[0]
    rd = ray_directions.reshape(n_rays, 3)
    out = pl.pallas_call(
        _body,
        grid=(n_rays // _R,),
        in_specs=[
            pl.BlockSpec((1, 3), lambda i: (0, 0)),
            pl.BlockSpec((_R, _N_STEPS), lambda i: (i, 0)),
            pl.BlockSpec((_R, _N_STEPS), lambda i: (i, 0)),
            pl.BlockSpec((_R, 3), lambda i: (i, 0)),
            pl.BlockSpec((_R, _N1), lambda i: (i, 0)),
        ],
        out_specs=pl.BlockSpec((_R, 3 * _N1), lambda i: (i, 0)),
        out_shape=jax.ShapeDtypeStruct((n_rays, 3 * _N1), jnp.float32),
    )(cam_loc, pts_intervals, occ_values, rd, tmp)
    return out.reshape(n_rays, _N1, 3)
